# baseline (speedup 1.0000x reference)
import functools
import jax
import jax.numpy as jnp
import numpy as np
from jax import lax
from jax.experimental import pallas as pl
from jax.experimental.pallas import tpu as pltpu
from jax.experimental.pallas import tpu_sc as plsc

N_NODES = 10000
E = 160000
C = 2048
NF = 39
EF = 2
HID = 384
LAT = 16
GD = 2
HEADS = 8
HD = 48
HF = HID // 2
NP = 48
KTEMP = 2.0

_INTERP = False


def _wfuse_body(w1, b1, w2, b2, g0w, g0b, g1w, g1b, g2w, g2b, atw,
                W0, B0, W1o, B1o, W2o, B2o, Wat, Bat):
    W0[...] = w2[...] @ g0w[...]
    B0[...] = b2[...] @ g0w[...] + g0b[...]
    W1o[...] = w2[...] @ g1w[...]
    B1o[...] = b2[...] @ g1w[...] + g1b[...]
    W2o[...] = w2[...] @ g2w[...]
    B2o[...] = b2[...] @ g2w[...] + g2b[...]
    Wat[...] = w2[...] @ atw[...]
    Bat[...] = b2[...] @ atw[...]


def _wfuse(p):
    g0w = jnp.pad(p['g0_ew'], ((0, 0), (0, NP - NF)))
    g0b = jnp.pad(p['g0_eb'], (0, NP - NF))[None]
    outs = pl.pallas_call(
        _wfuse_body,
        out_shape=[
            jax.ShapeDtypeStruct((HID, NP), jnp.float32),
            jax.ShapeDtypeStruct((1, NP), jnp.float32),
            jax.ShapeDtypeStruct((HID, HID), jnp.float32),
            jax.ShapeDtypeStruct((1, HID), jnp.float32),
            jax.ShapeDtypeStruct((HID, HID), jnp.float32),
            jax.ShapeDtypeStruct((1, HID), jnp.float32),
            jax.ShapeDtypeStruct((HID, HID), jnp.float32),
            jax.ShapeDtypeStruct((1, HID), jnp.float32),
        ],
        interpret=_INTERP,
    )(p['ee_w1'], p['ee_b1'][None], p['ee_w2'], p['ee_b2'][None],
      g0w, g0b, p['g1_ew'], p['g1_eb'][None], p['g2_ew'], p['g2_eb'][None],
      p['at_ew'])
    return outs


_BE = 2000


def _bmm(a, w):
    return jax.lax.dot_general(a.astype(jnp.bfloat16), w.astype(jnp.bfloat16),
                               (((1,), (0,)), ((), ())),
                               preferred_element_type=jnp.float32)


def _edge_mm_body(ea, w1, b1, W0, B0, Wat, Bat, el0, eef):
    r = jnp.maximum(ea[...] @ w1[...] + b1[...], 0.0)
    el0[...] = _bmm(r, W0[...]) + B0[...]
    eef[...] = _bmm(r, Wat[...]) + Bat[...]


def _edge_mm_q_body(ea, w1, b1, W1, B1, W2, B2, el1q, el2q):
    r = jnp.maximum(ea[...] @ w1[...] + b1[...], 0.0)
    el1q[...] = (_bmm(r, W1[0]) + B1[0])[None]
    el2q[...] = (_bmm(r, W2[0]) + B2[0])[None]


QW = 96


def _edge_mm(edge_attr, p, fw):
    W0, B0, W1, B1, W2, B2, Wat, Bat = fw
    ea = jnp.pad(edge_attr, ((0, 0), (0, 8 - EF)))
    w1 = jnp.pad(p['ee_w1'], ((0, 8 - EF), (0, 0)))
    nblk = E // _BE
    row = lambda w: pl.BlockSpec((_BE, w), lambda i: (i, 0))
    full = lambda a, b: pl.BlockSpec((a, b), lambda i: (0, 0))
    el0, eef = pl.pallas_call(
        _edge_mm_body,
        grid=(nblk,),
        in_specs=[row(8), full(8, HID), full(1, HID),
                  full(HID, NP), full(1, NP), full(HID, HID), full(1, HID)],
        out_specs=[row(NP), row(HID)],
        out_shape=[jax.ShapeDtypeStruct((E, NP), jnp.float32),
                   jax.ShapeDtypeStruct((E, HID), jnp.float32)],
        interpret=_INTERP,
    )(ea, w1, p['ee_b1'][None], W0, B0, Wat, Bat)
    W1q = W1.reshape(HID, 4, QW).transpose(1, 0, 2)
    B1q = B1.reshape(1, 4, QW).transpose(1, 0, 2)
    W2q = W2.reshape(HID, 4, QW).transpose(1, 0, 2)
    B2q = B2.reshape(1, 4, QW).transpose(1, 0, 2)
    rowq = lambda: pl.BlockSpec((_BE, 8), lambda q, i: (i, 0))
    wq = lambda: pl.BlockSpec((1, HID, QW), lambda q, i: (q, 0, 0))
    bq = lambda: pl.BlockSpec((1, 1, QW), lambda q, i: (q, 0, 0))
    oq = lambda: pl.BlockSpec((1, _BE, QW), lambda q, i: (q, i, 0))
    el1q, el2q = pl.pallas_call(
        _edge_mm_q_body,
        grid=(4, nblk),
        in_specs=[rowq(), pl.BlockSpec((8, HID), lambda q, i: (0, 0)),
                  pl.BlockSpec((1, HID), lambda q, i: (0, 0)),
                  wq(), bq(), wq(), bq()],
        out_specs=[oq(), oq()],
        out_shape=[jax.ShapeDtypeStruct((4, E, QW), jnp.float32),
                   jax.ShapeDtypeStruct((4, E, QW), jnp.float32)],
        interpret=_INTERP,
    )(ea, w1, p['ee_b1'][None], W1q, B1q, W2q, B2q)
    return el0, el1q, el2q, eef


_BN = 1000


def _node_body(first, h, a0, a1, a2, a3, mw1, mb1, mw2, mb2, g, b,
               hf, h0, h1, h2, h3):
    if first:
        agg = a0[0] + a1[0]
    else:
        agg = jnp.concatenate([a0[0], a1[0], a2[0], a3[0]], axis=1)
    o = jnp.maximum((h[...] + agg) @ mw1[...] + mb1[...], 0.0)
    o = o @ mw2[...] + mb2[...]
    if not first:
        o = h[...] + o
    m = jnp.mean(o, axis=-1, keepdims=True)
    v = jnp.mean(jnp.square(o - m), axis=-1, keepdims=True)
    o = (o - m) / jnp.sqrt(v + 1e-5) * g[...] + b[...]
    hf[...] = o
    h0[...] = o[:, 0 * QW:1 * QW]
    h1[...] = o[:, 1 * QW:2 * QW]
    h2[...] = o[:, 2 * QW:3 * QW]
    h3[...] = o[:, 3 * QW:4 * QW]


def _node_update(i, h, aggs, p):
    first = (i == 0)
    mi = NP if first else HID
    mw1 = jnp.pad(p['g0_mw1'], ((0, NP - NF), (0, 0))) if first else p['g%d_mw1' % i]
    row = lambda w: pl.BlockSpec((_BN, w), lambda j: (j, 0))
    full = lambda a, b: pl.BlockSpec((a, b), lambda j: (0, 0))
    if first:
        agg_in = [aggs, aggs, aggs, aggs]
        agg_specs = [pl.BlockSpec((1, _BN, NP), lambda j: (0, j, 0)),
                     pl.BlockSpec((1, _BN, NP), lambda j: (1, j, 0)),
                     pl.BlockSpec((1, _BN, NP), lambda j: (0, j, 0)),
                     pl.BlockSpec((1, _BN, NP), lambda j: (1, j, 0))]
    else:
        qa, qb = aggs
        agg_in = [qa, qa, qb, qb]
        agg_specs = [pl.BlockSpec((1, _BN, QW), lambda j: (0, j, 0)),
                     pl.BlockSpec((1, _BN, QW), lambda j: (1, j, 0)),
                     pl.BlockSpec((1, _BN, QW), lambda j: (0, j, 0)),
                     pl.BlockSpec((1, _BN, QW), lambda j: (1, j, 0))]
    outs = pl.pallas_call(
        functools.partial(_node_body, first),
        grid=(N_NODES // _BN,),
        in_specs=[row(mi)] + agg_specs +
                 [full(mi, HID), full(1, HID), full(HID, HID), full(1, HID),
                  full(1, HID), full(1, HID)],
        out_specs=[row(HID), row(QW), row(QW), row(QW), row(QW)],
        out_shape=[jax.ShapeDtypeStruct((N_NODES, HID), jnp.float32)] +
                  [jax.ShapeDtypeStruct((N_NODES, QW), jnp.float32)] * 4,
        interpret=_INTERP,
    )(h, *agg_in, mw1, p['g%d_mb1' % i][None], p['g%d_mw2' % i],
      p['g%d_mb2' % i][None], p['ln%d_g' % i][None], p['ln%d_b' % i][None])
    return outs


def _attn_body(h, lw, lb, rw, rb, xlf, xrf, x0, x1, x2, x3):
    xl = h[...] @ lw[...] + lb[...]
    xlf[...] = xl
    x0[...] = xl[:, 0 * QW:1 * QW]
    x1[...] = xl[:, 1 * QW:2 * QW]
    x2[...] = xl[:, 2 * QW:3 * QW]
    x3[...] = xl[:, 3 * QW:4 * QW]
    xrf[...] = h[...] @ rw[...] + rb[...]


def _attn_proj(h, p):
    row = lambda w: pl.BlockSpec((_BN, w), lambda j: (j, 0))
    full = lambda a, b: pl.BlockSpec((a, b), lambda j: (0, 0))
    return pl.pallas_call(
        _attn_body,
        grid=(N_NODES // _BN,),
        in_specs=[row(HID), full(HID, HID), full(1, HID),
                  full(HID, HID), full(1, HID)],
        out_specs=[row(HID), row(HID), row(QW), row(QW), row(QW), row(QW)],
        out_shape=[jax.ShapeDtypeStruct((N_NODES, HID), jnp.float32)] * 2 +
                  [jax.ShapeDtypeStruct((N_NODES, QW), jnp.float32)] * 4,
        interpret=_INTERP,
    )(h, p['at_lw'], p['at_lb'][None], p['at_rw'], p['at_rb'][None])


def _cden_body(d0, c16):
    c16[...] = KTEMP * jnp.log(d0[...])


def _cden(den0):
    return pl.pallas_call(
        _cden_body,
        out_shape=jax.ShapeDtypeStruct((N_NODES, 16), jnp.float32),
        interpret=_INTERP,
    )(den0)


def _gatfin_body(h, g0, g1, g2, g3, bias, gam, bet, h4):
    acc = jnp.concatenate([g0[0][:, :QW], g1[0][:, :QW],
                           g2[0][:, :QW], g3[0][:, :QW]], axis=1)
    d = g0[0][:, QW:QW + HEADS] + 1e-16
    d = jnp.repeat(d[:, :, None], HD, axis=2).reshape(acc.shape[0], HID)
    o = h[...] + acc / d + bias[...]
    m = jnp.mean(o, axis=-1, keepdims=True)
    v = jnp.mean(jnp.square(o - m), axis=-1, keepdims=True)
    h4[...] = (o - m) / jnp.sqrt(v + 1e-5) * gam[...] + bet[...]


def _gat_finish(h, gA, gB, p):
    row = lambda w: pl.BlockSpec((_BN, w), lambda j: (j, 0))
    full = lambda a, b: pl.BlockSpec((a, b), lambda j: (0, 0))
    q = lambda arr_i: pl.BlockSpec((1, _BN, _WD), lambda j, _i=arr_i: (_i, j, 0))
    return pl.pallas_call(
        _gatfin_body,
        grid=(N_NODES // _BN,),
        in_specs=[row(HID), q(0), q(1), q(0), q(1), full(1, HID),
                  full(1, HID), full(1, HID)],
        out_specs=row(HID),
        out_shape=jax.ShapeDtypeStruct((N_NODES, HID), jnp.float32),
        interpret=_INTERP,
    )(h, gA, gA, gB, gB, p['at_bias'][None], p['ln3_g'][None], p['ln3_b'][None])


def _head_body(cv, xg, gf, muw, mub, lvw, lvb, gmw, gmb,
               w1, b1, w2, b2, w3, b3, out, mu, logvar):
    cvv = cv[...]
    mu_ = cvv @ muw[...] + mub[...]
    mu[...] = mu_
    logvar[...] = cvv @ lvw[...] + lvb[...]
    su = xg[...][:, :33]
    elem = gf[...][:, 33:NF]
    esum = jnp.clip(jnp.sum(elem, axis=1, keepdims=True), 1.0, None)
    elem = jnp.where(jnp.max(elem) > 1.1, elem / esum, elem)
    g = 0.02 * jnp.maximum(elem @ gmw[...] + gmb[...], 0.0)
    xin = jnp.concatenate([su, mu_, g], axis=-1)
    o = jnp.maximum(xin @ w1[...] + b1[...], 0.0)
    o = jnp.maximum(o @ w2[...] + b2[...], 0.0)
    o = o @ w3[...] + b3[...]
    mu_pred = o[:, 0:1]
    x1 = o[:, 1:2]
    pi = jnp.maximum(x1, 0.0) + jnp.log1p(jnp.exp(-jnp.abs(x1)))
    out[...] = jnp.concatenate([mu_pred, pi], axis=1)


def _head(cv, xg, global_feat, p):
    return pl.pallas_call(
        _head_body,
        out_shape=[
            jax.ShapeDtypeStruct((C, 2), jnp.float32),
            jax.ShapeDtypeStruct((C, LAT), jnp.float32),
            jax.ShapeDtypeStruct((C, LAT), jnp.float32),
        ],
        interpret=_INTERP,
    )(cv, xg, global_feat, p['mu_w'], p['mu_b'][None], p['lv_w'],
      p['lv_b'][None], p['gm_w'], p['gm_b'][None],
      jnp.pad(p['d_w1'], ((0, 0), (0, 0))), p['d_b1'][None],
      p['d_w2'], p['d_b2'][None], p['d_w3'], p['d_b3'][None])


_SC_MESH = dict(core_axis_name="c", subcore_axis_name="s")


def _perm(v, idx):
    return lax.gather(
        v, idx[:, None],
        lax.GatherDimensionNumbers(offset_dims=(), collapsed_slice_dims=(0,),
                                   start_index_map=(0,)),
        (1,), mode=lax.GatherScatterMode.PROMISE_IN_BOUNDS)


def _lane_sum_all(u):
    io = lax.iota(jnp.int32, 16)
    for k in (8, 4, 2, 1):
        u = u + _perm(u, jnp.bitwise_xor(io, k))
    return u


def _zero_rows(buf, rows, w):
    z = jnp.zeros((16,), jnp.float32)

    def zb(e, _):
        for k in range(w // 16):
            buf[e, pl.ds(k * 16, 16)] = z
        return 0
    lax.fori_loop(0, rows, zb, 0)


_ZR = 8
_NCHUNK = N_NODES // _ZR


def _sc_epilogue(acc, out_ref, c, s, bounce, w):
    plsc.subcore_barrier()

    def body(j, _):
        cb = j * 16 + s

        @pl.when(cb < _NCHUNK)
        def _():
            r0 = cb * _ZR
            pltpu.sync_copy(acc.at[pl.ds(r0, _ZR)], bounce)
            pltpu.sync_copy(bounce, out_ref.at[c, pl.ds(r0, _ZR)])
        return 0
    lax.fori_loop(0, (_NCHUNK + 15) // 16, body, 0)


def _sc_prologue(acc, c, s, zbuf, w):
    _zero_rows(zbuf, _ZR, w)

    def body(j, _):
        cb = j * 16 + s

        @pl.when(cb < _NCHUNK)
        def _():
            pltpu.sync_copy(zbuf, acc.at[pl.ds(cb * _ZR, _ZR)])
        return 0
    lax.fori_loop(0, (_NCHUNK + 15) // 16, body, 0)
    plsc.subcore_barrier()


def _gine_seg_sum(hs, els, src, dst, width, qcall):
    col_split = qcall is not None
    ch = 80 if col_split else 40
    ept = E // 16 if col_split else E // 32
    n = ept // ch
    mesh = plsc.VectorSubcoreMesh(**_SC_MESH)
    slot = lambda t: [t, t]

    @functools.partial(
        pl.kernel, mesh=mesh,
        compiler_params=pltpu.CompilerParams(use_tc_tiling_on_sc=False),
        interpret=_INTERP,
        out_type=jax.ShapeDtypeStruct((2, N_NODES, width), jnp.float32),
        scratch_types=[
            slot(pltpu.VMEM((ch,), jnp.int32)),
            slot(pltpu.VMEM((ch,), jnp.int32)),
            slot(pltpu.VMEM((ch, width), jnp.float32)),
            slot(pltpu.VMEM((ch, width), jnp.float32)),
            pltpu.VMEM((_ZR, width), jnp.float32),
            pltpu.VMEM_SHARED((N_NODES, width), jnp.float32),
            slot(pltpu.SemaphoreType.DMA),
            slot(pltpu.SemaphoreType.DMA),
            slot(pltpu.SemaphoreType.DMA),
            slot(pltpu.SemaphoreType.DMA),
        ],
    )
    def k(hs_h, els_h, src_h, dst_h, out_h, idx_s, idx_d, hbuf, ebuf,
          zbuf, acc, sis, sid, sg, se):
        c = lax.axis_index("c")
        s = lax.axis_index("s")
        _sc_prologue(acc, c, s, zbuf, width)
        if col_split:
            tbase = s * ept
            q = 2 * qcall + c
            goff = q * N_NODES
        else:
            tbase = (c * 16 + s) * ept
            q = c * 0
            goff = 0

        def cbase(i):
            return tbase + i * ch

        def issue_idx(i, b):
            pltpu.async_copy(src_h.at[pl.ds(cbase(i), ch)], idx_s[b], sis[b])
            pltpu.async_copy(dst_h.at[pl.ds(cbase(i), ch)], idx_d[b], sid[b])

        def wait_idx(i, b):
            pltpu.make_async_copy(src_h.at[pl.ds(cbase(i), ch)], idx_s[b],
                                  sis[b]).wait()
            pltpu.make_async_copy(dst_h.at[pl.ds(cbase(i), ch)], idx_d[b],
                                  sid[b]).wait()
            if col_split:
                for j in range(ch // 16):
                    sl = pl.ds(j * 16, 16)
                    idx_s[b][sl] = idx_s[b][sl] + goff

        def issue_loads(i, b):
            pltpu.async_copy(hs_h.at[idx_s[b]], hbuf[b], sg[b])
            pltpu.async_copy(els_h.at[q, pl.ds(cbase(i), ch)], ebuf[b], se[b])

        def wait_loads(i, b):
            pltpu.make_async_copy(hs_h.at[idx_s[b]], hbuf[b], sg[b]).wait()
            pltpu.make_async_copy(els_h.at[q, pl.ds(cbase(i), ch)], ebuf[b],
                                  se[b]).wait()

        def compute(b):
            def one(e):
                for kk in range(width // 16):
                    sl = pl.ds(kk * 16, 16)
                    hbuf[b][e, sl] = jnp.maximum(
                        hbuf[b][e, sl] + ebuf[b][e, sl], 0.0)

            def ce(g, _):
                one(2 * g)
                one(2 * g + 1)
                return 0
            lax.fori_loop(0, ch // 2, ce, 0)

        issue_idx(0, 0)
        issue_idx(1, 1)
        wait_idx(0, 0)
        issue_loads(0, 0)

        def step(i, b, ob):
            wait_loads(i, b)

            @pl.when(i + 1 < n)
            def _():
                wait_idx(i + 1, ob)
                issue_loads(i + 1, ob)
            compute(b)
            pltpu.sync_copy(hbuf[b], acc.at[idx_d[b]], add=True)

            @pl.when(i + 2 < n)
            def _():
                issue_idx(i + 2, b)

        def pair(g, _):
            step(2 * g, 0, 1)
            step(2 * g + 1, 1, 0)
            return 0
        lax.fori_loop(0, n // 2, pair, 0)
        if n % 2:
            step(n - 1, 0, 1)
        _sc_epilogue(acc, out_h, c, s, zbuf, width)

    return k(hs, els, src, dst)


def _gat_alpha_sc(xlf, xrf, eef, src, dst, att):
    ch = 40
    ept = E // 32
    n = ept // ch
    mesh = plsc.VectorSubcoreMesh(**_SC_MESH)
    slot = lambda t: [t, t]

    @functools.partial(
        pl.kernel, mesh=mesh,
        compiler_params=pltpu.CompilerParams(use_tc_tiling_on_sc=False),
        interpret=_INTERP,
        out_type=[jax.ShapeDtypeStruct((E, 16), jnp.float32),
                  jax.ShapeDtypeStruct((2, N_NODES, 16), jnp.float32)],
        scratch_types=[
            slot(pltpu.VMEM((ch,), jnp.int32)),
            slot(pltpu.VMEM((ch,), jnp.int32)),
            slot(pltpu.VMEM((ch, HID), jnp.float32)),
            slot(pltpu.VMEM((ch, HID), jnp.float32)),
            slot(pltpu.VMEM((ch, HID), jnp.float32)),
            pltpu.VMEM((ch, 16), jnp.float32),
            pltpu.VMEM((ch, 16), jnp.float32),
            pltpu.VMEM((HID,), jnp.float32),
            pltpu.VMEM((_ZR, 16), jnp.float32),
            pltpu.VMEM_SHARED((N_NODES, 16), jnp.float32),
            slot(pltpu.SemaphoreType.DMA),
            slot(pltpu.SemaphoreType.DMA),
            slot(pltpu.SemaphoreType.DMA),
            slot(pltpu.SemaphoreType.DMA),
            slot(pltpu.SemaphoreType.DMA),
        ],
    )
    def k(xl_h, xr_h, ee_h, src_h, dst_h, att_h, alpha_h, den0_h,
          idx_s, idx_d, g1, g2, e3, abuf, dbuf, attb, zbuf, acc,
          sis, sid, sg1, sg2, se):
        c = lax.axis_index("c")
        s = lax.axis_index("s")
        pltpu.sync_copy(att_h, attb)
        _sc_prologue(acc, c, s, zbuf, 16)
        tbase = (c * 16 + s) * ept
        io = lax.iota(jnp.int32, 16)

        def cbase(i):
            return tbase + i * ch

        def issue_idx(i, b):
            pltpu.async_copy(src_h.at[pl.ds(cbase(i), ch)], idx_s[b], sis[b])
            pltpu.async_copy(dst_h.at[pl.ds(cbase(i), ch)], idx_d[b], sid[b])

        def wait_idx(i, b):
            pltpu.make_async_copy(src_h.at[pl.ds(cbase(i), ch)], idx_s[b],
                                  sis[b]).wait()
            pltpu.make_async_copy(dst_h.at[pl.ds(cbase(i), ch)], idx_d[b],
                                  sid[b]).wait()

        def issue_loads(i, b):
            pltpu.async_copy(xl_h.at[idx_s[b]], g1[b], sg1[b])
            pltpu.async_copy(xr_h.at[idx_d[b]], g2[b], sg2[b])
            pltpu.async_copy(ee_h.at[pl.ds(cbase(i), ch)], e3[b], se[b])

        def wait_loads(i, b):
            pltpu.make_async_copy(xl_h.at[idx_s[b]], g1[b], sg1[b]).wait()
            pltpu.make_async_copy(xr_h.at[idx_d[b]], g2[b], sg2[b]).wait()
            pltpu.make_async_copy(ee_h.at[pl.ds(cbase(i), ch)], e3[b],
                                  se[b]).wait()

        def compute(b):
            def one(e):
                us = []
                for h in range(HEADS):
                    u = None
                    for j in range(3):
                        kk = 3 * h + j
                        sl = pl.ds(kk * 16, 16)
                        v = g1[b][e, sl] + g2[b][e, sl] + e3[b][e, sl]
                        v = jnp.where(v > 0, v, 0.2 * v)
                        t = v * attb[sl]
                        u = t if u is None else u + t
                    us.append(u)
                sums = [_lane_sum_all(u) for u in us]
                row = jnp.zeros((16,), jnp.float32)
                for h in range(HEADS):
                    row = jnp.where(io == h, sums[h], row)
                abuf[e] = row
                dbuf[e] = jnp.exp(row * (1.0 / KTEMP))

            def ce(g, _):
                one(2 * g)
                one(2 * g + 1)
                return 0
            lax.fori_loop(0, ch // 2, ce, 0)

        issue_idx(0, 0)
        issue_idx(1, 1)
        wait_idx(0, 0)
        issue_loads(0, 0)

        def step(i, b, ob):
            wait_loads(i, b)

            @pl.when(i + 1 < n)
            def _():
                wait_idx(i + 1, ob)
                issue_loads(i + 1, ob)
            compute(b)
            pltpu.sync_copy(abuf, alpha_h.at[pl.ds(cbase(i), ch)])
            pltpu.sync_copy(dbuf, acc.at[idx_d[b]], add=True)

            @pl.when(i + 2 < n)
            def _():
                issue_idx(i + 2, b)

        def pair(g, _):
            step(2 * g, 0, 1)
            step(2 * g + 1, 1, 0)
            return 0
        lax.fori_loop(0, n // 2, pair, 0)
        if n % 2:
            step(n - 1, 0, 1)
        _sc_epilogue(acc, den0_h, c, s, zbuf, 16)

    return k(xlf, xrf, eef, src, dst, att)


_WD = QW + 16


def _gat_agg_sc(xlq, alpha, c16, src, dst, qcall):
    ch = 80
    ept = E // 16
    n = ept // ch
    mesh = plsc.VectorSubcoreMesh(**_SC_MESH)
    slot = lambda t: [t, t]

    @functools.partial(
        pl.kernel, mesh=mesh,
        compiler_params=pltpu.CompilerParams(use_tc_tiling_on_sc=False),
        interpret=_INTERP,
        out_type=jax.ShapeDtypeStruct((2, N_NODES, _WD), jnp.float32),
        scratch_types=[
            slot(pltpu.VMEM((ch,), jnp.int32)),
            slot(pltpu.VMEM((ch,), jnp.int32)),
            slot(pltpu.VMEM((ch, QW), jnp.float32)),
            slot(pltpu.VMEM((ch, 16), jnp.float32)),
            slot(pltpu.VMEM((ch, 16), jnp.float32)),
            pltpu.VMEM((ch, _WD), jnp.float32),
            pltpu.VMEM((_ZR, _WD), jnp.float32),
            pltpu.VMEM_SHARED((N_NODES, _WD), jnp.float32),
            slot(pltpu.SemaphoreType.DMA),
            slot(pltpu.SemaphoreType.DMA),
            slot(pltpu.SemaphoreType.DMA),
            slot(pltpu.SemaphoreType.DMA),
            slot(pltpu.SemaphoreType.DMA),
        ],
    )
    def k(xl_h, al_h, c16_h, src_h, dst_h, out_h,
          idx_s, idx_d, g1, abuf, cbuf, mbuf, zbuf, acc,
          sis, sid, sg1, sg2, se):
        c = lax.axis_index("c")
        s = lax.axis_index("s")
        _sc_prologue(acc, c, s, zbuf, _WD)
        tbase = s * ept
        q = 2 * qcall + c
        goff = q * N_NODES
        h2 = 2 * q

        def cbase(i):
            return tbase + i * ch

        def issue_idx(i, b):
            pltpu.async_copy(src_h.at[pl.ds(cbase(i), ch)], idx_s[b], sis[b])
            pltpu.async_copy(dst_h.at[pl.ds(cbase(i), ch)], idx_d[b], sid[b])

        def wait_idx(i, b):
            pltpu.make_async_copy(src_h.at[pl.ds(cbase(i), ch)], idx_s[b],
                                  sis[b]).wait()
            pltpu.make_async_copy(dst_h.at[pl.ds(cbase(i), ch)], idx_d[b],
                                  sid[b]).wait()
            for j in range(ch // 16):
                sl = pl.ds(j * 16, 16)
                idx_s[b][sl] = idx_s[b][sl] + goff

        def issue_loads(i, b):
            pltpu.async_copy(c16_h.at[idx_d[b]], cbuf[b], sg2[b])
            pltpu.async_copy(xl_h.at[idx_s[b]], g1[b], sg1[b])
            pltpu.async_copy(al_h.at[pl.ds(cbase(i), ch)], abuf[b], se[b])

        def wait_loads(i, b):
            pltpu.make_async_copy(c16_h.at[idx_d[b]], cbuf[b], sg2[b]).wait()
            pltpu.make_async_copy(xl_h.at[idx_s[b]], g1[b], sg1[b]).wait()
            pltpu.make_async_copy(al_h.at[pl.ds(cbase(i), ch)], abuf[b],
                                  se[b]).wait()

        def compute(b):
            def one(e):
                ex = jnp.exp(abuf[b][e] - cbuf[b][e])
                mbuf[e, pl.ds(QW, 16)] = ex
                for j in range(2):
                    bc = _perm(ex, jnp.full((16,), h2 + j, jnp.int32))
                    for t in range(3):
                        kk = j * 3 + t
                        sl = pl.ds(kk * 16, 16)
                        mbuf[e, sl] = g1[b][e, sl] * bc

            def ce(g, _):
                one(2 * g)
                one(2 * g + 1)
                return 0
            lax.fori_loop(0, ch // 2, ce, 0)

        issue_idx(0, 0)
        issue_idx(1, 1)
        wait_idx(0, 0)
        issue_loads(0, 0)

        def step(i, b, ob):
            wait_loads(i, b)

            @pl.when(i + 1 < n)
            def _():
                wait_idx(i + 1, ob)
                issue_loads(i + 1, ob)
            compute(b)
            pltpu.sync_copy(mbuf, acc.at[idx_d[b]], add=True)

            @pl.when(i + 2 < n)
            def _():
                issue_idx(i + 2, b)

        def pair(g, _):
            step(2 * g, 0, 1)
            step(2 * g + 1, 1, 0)
            return 0
        lax.fori_loop(0, n // 2, pair, 0)
        if n % 2:
            step(n - 1, 0, 1)
        _sc_epilogue(acc, out_h, c, s, zbuf, _WD)

    return k(xlq, alpha, c16, src, dst)


def _center_gather_sc(h4, x_pad, center_id):
    ch = C // 32
    mesh = plsc.VectorSubcoreMesh(**_SC_MESH)

    @functools.partial(
        pl.kernel, mesh=mesh,
        compiler_params=pltpu.CompilerParams(use_tc_tiling_on_sc=False),
        interpret=_INTERP,
        out_type=[jax.ShapeDtypeStruct((C, HID), jnp.float32),
                  jax.ShapeDtypeStruct((C, NP), jnp.float32)],
        scratch_types=[
            pltpu.VMEM((ch,), jnp.int32),
            pltpu.VMEM((ch, HID), jnp.float32),
            pltpu.VMEM((ch, NP), jnp.float32),
            pltpu.SemaphoreType.DMA,
        ],
    )
    def k(h4_h, xp_h, cid_h, cv_h, xg_h, idx, buf1, buf2, sem):
        c = lax.axis_index("c")
        s = lax.axis_index("s")
        base = (c * 16 + s) * ch
        pltpu.sync_copy(cid_h.at[pl.ds(base, ch)], idx)
        pltpu.async_copy(h4_h.at[idx], buf1, sem).wait()
        pltpu.sync_copy(buf1, cv_h.at[pl.ds(base, ch)])
        pltpu.async_copy(xp_h.at[idx], buf2, sem).wait()
        pltpu.sync_copy(buf2, xg_h.at[pl.ds(base, ch)])

    return k(h4, x_pad, center_id)


def kernel(x, edge_attr, global_feat, params, edge_index, center_id):
    p = params
    src = edge_index[0]
    dst = edge_index[1]
    x_pad = jnp.pad(x, ((0, 0), (0, NP - NF)))

    fw = _wfuse(p)
    el0, el1q, el2q, eef = _edge_mm(edge_attr, p, fw)

    agg0 = _gine_seg_sum(x_pad, el0[None], src, dst, NP, qcall=None)
    h, h0, h1, h2, h3 = _node_update(0, x_pad, agg0, p)

    for i, elq in ((1, el1q), (2, el2q)):
        hq = jnp.concatenate([h0, h1, h2, h3], axis=0)
        qa = _gine_seg_sum(hq, elq, src, dst, QW, qcall=0)
        qb = _gine_seg_sum(hq, elq, src, dst, QW, qcall=1)
        h, h0, h1, h2, h3 = _node_update(i, h, (qa, qb), p)

    xlf, xrf, x0, x1, x2, x3 = _attn_proj(h, p)
    alpha16, den0 = _gat_alpha_sc(xlf, xrf, eef, src, dst,
                                  p['at_att'].reshape(-1))
    c16 = _cden(den0[0] + den0[1])
    xlq = jnp.concatenate([x0, x1, x2, x3], axis=0)
    gA = _gat_agg_sc(xlq, alpha16, c16, src, dst, qcall=0)
    gB = _gat_agg_sc(xlq, alpha16, c16, src, dst, qcall=1)
    h4 = _gat_finish(h, gA, gB, p)

    cv, xg = _center_gather_sc(h4, x_pad, center_id)
    out, mu, logvar = _head(cv, xg, global_feat, p)
    return ((out[:, 0], out[:, 1]), (mu, logvar))


# speedup vs baseline: 1.0155x; 1.0155x over previous
import functools
import jax
import jax.numpy as jnp
import numpy as np
from jax import lax
from jax.experimental import pallas as pl
from jax.experimental.pallas import tpu as pltpu
from jax.experimental.pallas import tpu_sc as plsc

N_NODES = 10000
E = 160000
C = 2048
NF = 39
EF = 2
HID = 384
LAT = 16
GD = 2
HEADS = 8
HD = 48
HF = HID // 2
NP = 48
KTEMP = 2.0

_INTERP = False


def _wfuse_body(w1, b1, w2, b2, g0w, g0b, g1w, g1b, g2w, g2b, atw,
                W0, B0, W1o, B1o, W2o, B2o, Wat, Bat):
    W0[...] = w2[...] @ g0w[...]
    B0[...] = b2[...] @ g0w[...] + g0b[...]
    W1o[...] = w2[...] @ g1w[...]
    B1o[...] = b2[...] @ g1w[...] + g1b[...]
    W2o[...] = w2[...] @ g2w[...]
    B2o[...] = b2[...] @ g2w[...] + g2b[...]
    Wat[...] = w2[...] @ atw[...]
    Bat[...] = b2[...] @ atw[...]


def _wfuse(p):
    g0w = jnp.pad(p['g0_ew'], ((0, 0), (0, NP - NF)))
    g0b = jnp.pad(p['g0_eb'], (0, NP - NF))[None]
    outs = pl.pallas_call(
        _wfuse_body,
        out_shape=[
            jax.ShapeDtypeStruct((HID, NP), jnp.float32),
            jax.ShapeDtypeStruct((1, NP), jnp.float32),
            jax.ShapeDtypeStruct((HID, HID), jnp.float32),
            jax.ShapeDtypeStruct((1, HID), jnp.float32),
            jax.ShapeDtypeStruct((HID, HID), jnp.float32),
            jax.ShapeDtypeStruct((1, HID), jnp.float32),
            jax.ShapeDtypeStruct((HID, HID), jnp.float32),
            jax.ShapeDtypeStruct((1, HID), jnp.float32),
        ],
        interpret=_INTERP,
    )(p['ee_w1'], p['ee_b1'][None], p['ee_w2'], p['ee_b2'][None],
      g0w, g0b, p['g1_ew'], p['g1_eb'][None], p['g2_ew'], p['g2_eb'][None],
      p['at_ew'])
    return outs


_BE = 2000


def _edge_mm_body(ea, w1, b1, W0, B0, Wat, Bat, el0, eef):
    r = jnp.maximum(ea[...] @ w1[...] + b1[...], 0.0)
    el0[...] = r @ W0[...] + B0[...]
    eef[...] = r @ Wat[...] + Bat[...]


def _edge_mm_q_body(ea, w1, b1, W1, B1, W2, B2, el1q, el2q):
    r = jnp.maximum(ea[...] @ w1[...] + b1[...], 0.0)
    el1q[...] = (r @ W1[0] + B1[0])[None]
    el2q[...] = (r @ W2[0] + B2[0])[None]


QW = 96


def _edge_mm(edge_attr, p, fw):
    W0, B0, W1, B1, W2, B2, Wat, Bat = fw
    ea = jnp.pad(edge_attr, ((0, 0), (0, 8 - EF)))
    w1 = jnp.pad(p['ee_w1'], ((0, 8 - EF), (0, 0)))
    nblk = E // _BE
    row = lambda w: pl.BlockSpec((_BE, w), lambda i: (i, 0))
    full = lambda a, b: pl.BlockSpec((a, b), lambda i: (0, 0))
    el0, eef = pl.pallas_call(
        _edge_mm_body,
        grid=(nblk,),
        in_specs=[row(8), full(8, HID), full(1, HID),
                  full(HID, NP), full(1, NP), full(HID, HID), full(1, HID)],
        out_specs=[row(NP), row(HID)],
        out_shape=[jax.ShapeDtypeStruct((E, NP), jnp.float32),
                   jax.ShapeDtypeStruct((E, HID), jnp.float32)],
        interpret=_INTERP,
    )(ea, w1, p['ee_b1'][None], W0, B0, Wat, Bat)
    W1q = W1.reshape(HID, 4, QW).transpose(1, 0, 2)
    B1q = B1.reshape(1, 4, QW).transpose(1, 0, 2)
    W2q = W2.reshape(HID, 4, QW).transpose(1, 0, 2)
    B2q = B2.reshape(1, 4, QW).transpose(1, 0, 2)
    rowq = lambda: pl.BlockSpec((_BE, 8), lambda q, i: (i, 0))
    wq = lambda: pl.BlockSpec((1, HID, QW), lambda q, i: (q, 0, 0))
    bq = lambda: pl.BlockSpec((1, 1, QW), lambda q, i: (q, 0, 0))
    oq = lambda: pl.BlockSpec((1, _BE, QW), lambda q, i: (q, i, 0))
    el1q, el2q = pl.pallas_call(
        _edge_mm_q_body,
        grid=(4, nblk),
        in_specs=[rowq(), pl.BlockSpec((8, HID), lambda q, i: (0, 0)),
                  pl.BlockSpec((1, HID), lambda q, i: (0, 0)),
                  wq(), bq(), wq(), bq()],
        out_specs=[oq(), oq()],
        out_shape=[jax.ShapeDtypeStruct((4, E, QW), jnp.float32),
                   jax.ShapeDtypeStruct((4, E, QW), jnp.float32)],
        interpret=_INTERP,
    )(ea, w1, p['ee_b1'][None], W1q, B1q, W2q, B2q)
    return el0, el1q, el2q, eef


_BN = 1000


def _node_body(first, h, a0, a1, a2, a3, mw1, mb1, mw2, mb2, g, b,
               hf, h0, h1, h2, h3):
    if first:
        agg = a0[0] + a1[0]
    else:
        agg = jnp.concatenate([a0[0], a1[0], a2[0], a3[0]], axis=1)
    o = jnp.maximum((h[...] + agg) @ mw1[...] + mb1[...], 0.0)
    o = o @ mw2[...] + mb2[...]
    if not first:
        o = h[...] + o
    m = jnp.mean(o, axis=-1, keepdims=True)
    v = jnp.mean(jnp.square(o - m), axis=-1, keepdims=True)
    o = (o - m) / jnp.sqrt(v + 1e-5) * g[...] + b[...]
    hf[...] = o
    h0[...] = o[:, 0 * QW:1 * QW]
    h1[...] = o[:, 1 * QW:2 * QW]
    h2[...] = o[:, 2 * QW:3 * QW]
    h3[...] = o[:, 3 * QW:4 * QW]


def _node_update(i, h, aggs, p):
    first = (i == 0)
    mi = NP if first else HID
    mw1 = jnp.pad(p['g0_mw1'], ((0, NP - NF), (0, 0))) if first else p['g%d_mw1' % i]
    row = lambda w: pl.BlockSpec((_BN, w), lambda j: (j, 0))
    full = lambda a, b: pl.BlockSpec((a, b), lambda j: (0, 0))
    if first:
        agg_in = [aggs, aggs, aggs, aggs]
        agg_specs = [pl.BlockSpec((1, _BN, NP), lambda j: (0, j, 0)),
                     pl.BlockSpec((1, _BN, NP), lambda j: (1, j, 0)),
                     pl.BlockSpec((1, _BN, NP), lambda j: (0, j, 0)),
                     pl.BlockSpec((1, _BN, NP), lambda j: (1, j, 0))]
    else:
        qa, qb = aggs
        agg_in = [qa, qa, qb, qb]
        agg_specs = [pl.BlockSpec((1, _BN, QW), lambda j: (0, j, 0)),
                     pl.BlockSpec((1, _BN, QW), lambda j: (1, j, 0)),
                     pl.BlockSpec((1, _BN, QW), lambda j: (0, j, 0)),
                     pl.BlockSpec((1, _BN, QW), lambda j: (1, j, 0))]
    outs = pl.pallas_call(
        functools.partial(_node_body, first),
        grid=(N_NODES // _BN,),
        in_specs=[row(mi)] + agg_specs +
                 [full(mi, HID), full(1, HID), full(HID, HID), full(1, HID),
                  full(1, HID), full(1, HID)],
        out_specs=[row(HID), row(QW), row(QW), row(QW), row(QW)],
        out_shape=[jax.ShapeDtypeStruct((N_NODES, HID), jnp.float32)] +
                  [jax.ShapeDtypeStruct((N_NODES, QW), jnp.float32)] * 4,
        interpret=_INTERP,
    )(h, *agg_in, mw1, p['g%d_mb1' % i][None], p['g%d_mw2' % i],
      p['g%d_mb2' % i][None], p['ln%d_g' % i][None], p['ln%d_b' % i][None])
    return outs


def _attn_body(h, lw, lb, rw, rb, xlf, xrf, x0, x1, x2, x3):
    xl = h[...] @ lw[...] + lb[...]
    xlf[...] = xl
    x0[...] = xl[:, 0 * QW:1 * QW]
    x1[...] = xl[:, 1 * QW:2 * QW]
    x2[...] = xl[:, 2 * QW:3 * QW]
    x3[...] = xl[:, 3 * QW:4 * QW]
    xrf[...] = h[...] @ rw[...] + rb[...]


def _attn_proj(h, p):
    row = lambda w: pl.BlockSpec((_BN, w), lambda j: (j, 0))
    full = lambda a, b: pl.BlockSpec((a, b), lambda j: (0, 0))
    return pl.pallas_call(
        _attn_body,
        grid=(N_NODES // _BN,),
        in_specs=[row(HID), full(HID, HID), full(1, HID),
                  full(HID, HID), full(1, HID)],
        out_specs=[row(HID), row(HID), row(QW), row(QW), row(QW), row(QW)],
        out_shape=[jax.ShapeDtypeStruct((N_NODES, HID), jnp.float32)] * 2 +
                  [jax.ShapeDtypeStruct((N_NODES, QW), jnp.float32)] * 4,
        interpret=_INTERP,
    )(h, p['at_lw'], p['at_lb'][None], p['at_rw'], p['at_rb'][None])


def _cden_body(d0, c16):
    c16[...] = KTEMP * jnp.log(d0[...])


def _cden(den0):
    return pl.pallas_call(
        _cden_body,
        out_shape=jax.ShapeDtypeStruct((N_NODES, 16), jnp.float32),
        interpret=_INTERP,
    )(den0)


def _gatfin_body(h, g0, g1, g2, g3, bias, gam, bet, h4):
    acc = jnp.concatenate([g0[0][:, :QW], g1[0][:, :QW],
                           g2[0][:, :QW], g3[0][:, :QW]], axis=1)
    d = g0[0][:, QW:QW + HEADS] + 1e-16
    d = jnp.repeat(d[:, :, None], HD, axis=2).reshape(acc.shape[0], HID)
    o = h[...] + acc / d + bias[...]
    m = jnp.mean(o, axis=-1, keepdims=True)
    v = jnp.mean(jnp.square(o - m), axis=-1, keepdims=True)
    h4[...] = (o - m) / jnp.sqrt(v + 1e-5) * gam[...] + bet[...]


def _gat_finish(h, gA, gB, p):
    row = lambda w: pl.BlockSpec((_BN, w), lambda j: (j, 0))
    full = lambda a, b: pl.BlockSpec((a, b), lambda j: (0, 0))
    q = lambda arr_i: pl.BlockSpec((1, _BN, _WD), lambda j, _i=arr_i: (_i, j, 0))
    return pl.pallas_call(
        _gatfin_body,
        grid=(N_NODES // _BN,),
        in_specs=[row(HID), q(0), q(1), q(0), q(1), full(1, HID),
                  full(1, HID), full(1, HID)],
        out_specs=row(HID),
        out_shape=jax.ShapeDtypeStruct((N_NODES, HID), jnp.float32),
        interpret=_INTERP,
    )(h, gA, gA, gB, gB, p['at_bias'][None], p['ln3_g'][None], p['ln3_b'][None])


def _head_body(cv, xg, gf, muw, mub, lvw, lvb, gmw, gmb,
               w1, b1, w2, b2, w3, b3, out, mu, logvar):
    cvv = cv[...]
    mu_ = cvv @ muw[...] + mub[...]
    mu[...] = mu_
    logvar[...] = cvv @ lvw[...] + lvb[...]
    su = xg[...][:, :33]
    elem = gf[...][:, 33:NF]
    esum = jnp.clip(jnp.sum(elem, axis=1, keepdims=True), 1.0, None)
    elem = jnp.where(jnp.max(elem) > 1.1, elem / esum, elem)
    g = 0.02 * jnp.maximum(elem @ gmw[...] + gmb[...], 0.0)
    xin = jnp.concatenate([su, mu_, g], axis=-1)
    o = jnp.maximum(xin @ w1[...] + b1[...], 0.0)
    o = jnp.maximum(o @ w2[...] + b2[...], 0.0)
    o = o @ w3[...] + b3[...]
    mu_pred = o[:, 0:1]
    x1 = o[:, 1:2]
    pi = jnp.maximum(x1, 0.0) + jnp.log1p(jnp.exp(-jnp.abs(x1)))
    out[...] = jnp.concatenate([mu_pred, pi], axis=1)


def _head(cv, xg, global_feat, p):
    return pl.pallas_call(
        _head_body,
        out_shape=[
            jax.ShapeDtypeStruct((C, 2), jnp.float32),
            jax.ShapeDtypeStruct((C, LAT), jnp.float32),
            jax.ShapeDtypeStruct((C, LAT), jnp.float32),
        ],
        interpret=_INTERP,
    )(cv, xg, global_feat, p['mu_w'], p['mu_b'][None], p['lv_w'],
      p['lv_b'][None], p['gm_w'], p['gm_b'][None],
      jnp.pad(p['d_w1'], ((0, 0), (0, 0))), p['d_b1'][None],
      p['d_w2'], p['d_b2'][None], p['d_w3'], p['d_b3'][None])


_SC_MESH = dict(core_axis_name="c", subcore_axis_name="s")


def _perm(v, idx):
    return lax.gather(
        v, idx[:, None],
        lax.GatherDimensionNumbers(offset_dims=(), collapsed_slice_dims=(0,),
                                   start_index_map=(0,)),
        (1,), mode=lax.GatherScatterMode.PROMISE_IN_BOUNDS)


def _lane_sum_all(u):
    io = lax.iota(jnp.int32, 16)
    for k in (8, 4, 2, 1):
        u = u + _perm(u, jnp.bitwise_xor(io, k))
    return u


def _zero_rows(buf, rows, w):
    z = jnp.zeros((16,), jnp.float32)

    def zb(e, _):
        for k in range(w // 16):
            buf[e, pl.ds(k * 16, 16)] = z
        return 0
    lax.fori_loop(0, rows, zb, 0)


_ZR = 8
_NCHUNK = N_NODES // _ZR


def _sc_epilogue(acc, out_ref, c, s, bounce, w):
    plsc.subcore_barrier()

    def body(j, _):
        cb = j * 16 + s

        @pl.when(cb < _NCHUNK)
        def _():
            r0 = cb * _ZR
            pltpu.sync_copy(acc.at[pl.ds(r0, _ZR)], bounce)
            pltpu.sync_copy(bounce, out_ref.at[c, pl.ds(r0, _ZR)])
        return 0
    lax.fori_loop(0, (_NCHUNK + 15) // 16, body, 0)


def _sc_prologue(acc, c, s, zbuf, w):
    _zero_rows(zbuf, _ZR, w)

    def body(j, _):
        cb = j * 16 + s

        @pl.when(cb < _NCHUNK)
        def _():
            pltpu.sync_copy(zbuf, acc.at[pl.ds(cb * _ZR, _ZR)])
        return 0
    lax.fori_loop(0, (_NCHUNK + 15) // 16, body, 0)
    plsc.subcore_barrier()


def _gine_seg_sum(hs, els, src, dst, width, qcall):
    col_split = qcall is not None
    ch = 80 if col_split else 40
    ept = E // 16 if col_split else E // 32
    n = ept // ch
    mesh = plsc.VectorSubcoreMesh(**_SC_MESH)
    slot = lambda t: [t, t]

    @functools.partial(
        pl.kernel, mesh=mesh,
        compiler_params=pltpu.CompilerParams(use_tc_tiling_on_sc=False),
        interpret=_INTERP,
        out_type=jax.ShapeDtypeStruct((2, N_NODES, width), jnp.float32),
        scratch_types=[
            slot(pltpu.VMEM((ch,), jnp.int32)),
            slot(pltpu.VMEM((ch,), jnp.int32)),
            slot(pltpu.VMEM((ch, width), jnp.float32)),
            slot(pltpu.VMEM((ch, width), jnp.float32)),
            pltpu.VMEM((_ZR, width), jnp.float32),
            pltpu.VMEM_SHARED((N_NODES, width), jnp.float32),
            slot(pltpu.SemaphoreType.DMA),
            slot(pltpu.SemaphoreType.DMA),
            slot(pltpu.SemaphoreType.DMA),
            slot(pltpu.SemaphoreType.DMA),
        ],
    )
    def k(hs_h, els_h, src_h, dst_h, out_h, idx_s, idx_d, hbuf, ebuf,
          zbuf, acc, sis, sid, sg, se):
        c = lax.axis_index("c")
        s = lax.axis_index("s")
        _sc_prologue(acc, c, s, zbuf, width)
        if col_split:
            tbase = s * ept
            q = 2 * qcall + c
            goff = q * N_NODES
        else:
            tbase = (c * 16 + s) * ept
            q = c * 0
            goff = 0

        def cbase(i):
            return tbase + i * ch

        def issue_idx(i, b):
            pltpu.async_copy(src_h.at[pl.ds(cbase(i), ch)], idx_s[b], sis[b])
            pltpu.async_copy(dst_h.at[pl.ds(cbase(i), ch)], idx_d[b], sid[b])

        def wait_idx(i, b):
            pltpu.make_async_copy(src_h.at[pl.ds(cbase(i), ch)], idx_s[b],
                                  sis[b]).wait()
            pltpu.make_async_copy(dst_h.at[pl.ds(cbase(i), ch)], idx_d[b],
                                  sid[b]).wait()
            if col_split:
                for j in range(ch // 16):
                    sl = pl.ds(j * 16, 16)
                    idx_s[b][sl] = idx_s[b][sl] + goff

        def issue_loads(i, b):
            pltpu.async_copy(hs_h.at[idx_s[b]], hbuf[b], sg[b])
            pltpu.async_copy(els_h.at[q, pl.ds(cbase(i), ch)], ebuf[b], se[b])

        def wait_loads(i, b):
            pltpu.make_async_copy(hs_h.at[idx_s[b]], hbuf[b], sg[b]).wait()
            pltpu.make_async_copy(els_h.at[q, pl.ds(cbase(i), ch)], ebuf[b],
                                  se[b]).wait()

        def compute(b):
            def ce(e, _):
                for kk in range(width // 16):
                    sl = pl.ds(kk * 16, 16)
                    hbuf[b][e, sl] = jnp.maximum(
                        hbuf[b][e, sl] + ebuf[b][e, sl], 0.0)
                return 0
            lax.fori_loop(0, ch, ce, 0)

        issue_idx(0, 0)
        issue_idx(1, 1)
        wait_idx(0, 0)
        issue_loads(0, 0)

        def step(i, b, ob):
            wait_loads(i, b)

            @pl.when(i + 1 < n)
            def _():
                wait_idx(i + 1, ob)
                issue_loads(i + 1, ob)
            compute(b)
            pltpu.sync_copy(hbuf[b], acc.at[idx_d[b]], add=True)

            @pl.when(i + 2 < n)
            def _():
                issue_idx(i + 2, b)

        def pair(g, _):
            step(2 * g, 0, 1)
            step(2 * g + 1, 1, 0)
            return 0
        lax.fori_loop(0, n // 2, pair, 0)
        if n % 2:
            step(n - 1, 0, 1)
        _sc_epilogue(acc, out_h, c, s, zbuf, width)

    return k(hs, els, src, dst)


def _gat_alpha_sc(xlf, xrf, eef, src, dst, att):
    ch = 40
    ept = E // 32
    n = ept // ch
    mesh = plsc.VectorSubcoreMesh(**_SC_MESH)
    slot = lambda t: [t, t]

    @functools.partial(
        pl.kernel, mesh=mesh,
        compiler_params=pltpu.CompilerParams(use_tc_tiling_on_sc=False),
        interpret=_INTERP,
        out_type=[jax.ShapeDtypeStruct((E, 16), jnp.float32),
                  jax.ShapeDtypeStruct((2, N_NODES, 16), jnp.float32)],
        scratch_types=[
            slot(pltpu.VMEM((ch,), jnp.int32)),
            slot(pltpu.VMEM((ch,), jnp.int32)),
            slot(pltpu.VMEM((ch, HID), jnp.float32)),
            slot(pltpu.VMEM((ch, HID), jnp.float32)),
            slot(pltpu.VMEM((ch, HID), jnp.float32)),
            pltpu.VMEM((ch, 16), jnp.float32),
            pltpu.VMEM((ch, 16), jnp.float32),
            pltpu.VMEM((HID,), jnp.float32),
            pltpu.VMEM((_ZR, 16), jnp.float32),
            pltpu.VMEM_SHARED((N_NODES, 16), jnp.float32),
            slot(pltpu.SemaphoreType.DMA),
            slot(pltpu.SemaphoreType.DMA),
            slot(pltpu.SemaphoreType.DMA),
            slot(pltpu.SemaphoreType.DMA),
            slot(pltpu.SemaphoreType.DMA),
        ],
    )
    def k(xl_h, xr_h, ee_h, src_h, dst_h, att_h, alpha_h, den0_h,
          idx_s, idx_d, g1, g2, e3, abuf, dbuf, attb, zbuf, acc,
          sis, sid, sg1, sg2, se):
        c = lax.axis_index("c")
        s = lax.axis_index("s")
        pltpu.sync_copy(att_h, attb)
        _sc_prologue(acc, c, s, zbuf, 16)
        tbase = (c * 16 + s) * ept
        io = lax.iota(jnp.int32, 16)

        def cbase(i):
            return tbase + i * ch

        def issue_idx(i, b):
            pltpu.async_copy(src_h.at[pl.ds(cbase(i), ch)], idx_s[b], sis[b])
            pltpu.async_copy(dst_h.at[pl.ds(cbase(i), ch)], idx_d[b], sid[b])

        def wait_idx(i, b):
            pltpu.make_async_copy(src_h.at[pl.ds(cbase(i), ch)], idx_s[b],
                                  sis[b]).wait()
            pltpu.make_async_copy(dst_h.at[pl.ds(cbase(i), ch)], idx_d[b],
                                  sid[b]).wait()

        def issue_loads(i, b):
            pltpu.async_copy(xl_h.at[idx_s[b]], g1[b], sg1[b])
            pltpu.async_copy(xr_h.at[idx_d[b]], g2[b], sg2[b])
            pltpu.async_copy(ee_h.at[pl.ds(cbase(i), ch)], e3[b], se[b])

        def wait_loads(i, b):
            pltpu.make_async_copy(xl_h.at[idx_s[b]], g1[b], sg1[b]).wait()
            pltpu.make_async_copy(xr_h.at[idx_d[b]], g2[b], sg2[b]).wait()
            pltpu.make_async_copy(ee_h.at[pl.ds(cbase(i), ch)], e3[b],
                                  se[b]).wait()

        def compute(b):
            def ce(e, _):
                us = []
                for h in range(HEADS):
                    u = None
                    for j in range(3):
                        kk = 3 * h + j
                        sl = pl.ds(kk * 16, 16)
                        v = g1[b][e, sl] + g2[b][e, sl] + e3[b][e, sl]
                        v = jnp.where(v > 0, v, 0.2 * v)
                        t = v * attb[sl]
                        u = t if u is None else u + t
                    us.append(u)
                sums = [_lane_sum_all(u) for u in us]
                row = jnp.zeros((16,), jnp.float32)
                for h in range(HEADS):
                    row = jnp.where(io == h, sums[h], row)
                abuf[e] = row
                dbuf[e] = jnp.exp(row * (1.0 / KTEMP))
                return 0
            lax.fori_loop(0, ch, ce, 0)

        issue_idx(0, 0)
        issue_idx(1, 1)
        wait_idx(0, 0)
        issue_loads(0, 0)

        def step(i, b, ob):
            wait_loads(i, b)

            @pl.when(i + 1 < n)
            def _():
                wait_idx(i + 1, ob)
                issue_loads(i + 1, ob)
            compute(b)
            pltpu.sync_copy(abuf, alpha_h.at[pl.ds(cbase(i), ch)])
            pltpu.sync_copy(dbuf, acc.at[idx_d[b]], add=True)

            @pl.when(i + 2 < n)
            def _():
                issue_idx(i + 2, b)

        def pair(g, _):
            step(2 * g, 0, 1)
            step(2 * g + 1, 1, 0)
            return 0
        lax.fori_loop(0, n // 2, pair, 0)
        if n % 2:
            step(n - 1, 0, 1)
        _sc_epilogue(acc, den0_h, c, s, zbuf, 16)

    return k(xlf, xrf, eef, src, dst, att)


_WD = QW + 16


def _gat_agg_sc(xlq, alpha, c16, src, dst, qcall):
    ch = 80
    ept = E // 16
    n = ept // ch
    mesh = plsc.VectorSubcoreMesh(**_SC_MESH)
    slot = lambda t: [t, t]

    @functools.partial(
        pl.kernel, mesh=mesh,
        compiler_params=pltpu.CompilerParams(use_tc_tiling_on_sc=False),
        interpret=_INTERP,
        out_type=jax.ShapeDtypeStruct((2, N_NODES, _WD), jnp.float32),
        scratch_types=[
            slot(pltpu.VMEM((ch,), jnp.int32)),
            slot(pltpu.VMEM((ch,), jnp.int32)),
            slot(pltpu.VMEM((ch, QW), jnp.float32)),
            slot(pltpu.VMEM((ch, 16), jnp.float32)),
            slot(pltpu.VMEM((ch, 16), jnp.float32)),
            pltpu.VMEM((ch, _WD), jnp.float32),
            pltpu.VMEM((_ZR, _WD), jnp.float32),
            pltpu.VMEM_SHARED((N_NODES, _WD), jnp.float32),
            slot(pltpu.SemaphoreType.DMA),
            slot(pltpu.SemaphoreType.DMA),
            slot(pltpu.SemaphoreType.DMA),
            slot(pltpu.SemaphoreType.DMA),
            slot(pltpu.SemaphoreType.DMA),
        ],
    )
    def k(xl_h, al_h, c16_h, src_h, dst_h, out_h,
          idx_s, idx_d, g1, abuf, cbuf, mbuf, zbuf, acc,
          sis, sid, sg1, sg2, se):
        c = lax.axis_index("c")
        s = lax.axis_index("s")
        _sc_prologue(acc, c, s, zbuf, _WD)
        tbase = s * ept
        q = 2 * qcall + c
        goff = q * N_NODES
        h2 = 2 * q

        def cbase(i):
            return tbase + i * ch

        def issue_idx(i, b):
            pltpu.async_copy(src_h.at[pl.ds(cbase(i), ch)], idx_s[b], sis[b])
            pltpu.async_copy(dst_h.at[pl.ds(cbase(i), ch)], idx_d[b], sid[b])

        def wait_idx(i, b):
            pltpu.make_async_copy(src_h.at[pl.ds(cbase(i), ch)], idx_s[b],
                                  sis[b]).wait()
            pltpu.make_async_copy(dst_h.at[pl.ds(cbase(i), ch)], idx_d[b],
                                  sid[b]).wait()
            for j in range(ch // 16):
                sl = pl.ds(j * 16, 16)
                idx_s[b][sl] = idx_s[b][sl] + goff

        def issue_loads(i, b):
            pltpu.async_copy(c16_h.at[idx_d[b]], cbuf[b], sg2[b])
            pltpu.async_copy(xl_h.at[idx_s[b]], g1[b], sg1[b])
            pltpu.async_copy(al_h.at[pl.ds(cbase(i), ch)], abuf[b], se[b])

        def wait_loads(i, b):
            pltpu.make_async_copy(c16_h.at[idx_d[b]], cbuf[b], sg2[b]).wait()
            pltpu.make_async_copy(xl_h.at[idx_s[b]], g1[b], sg1[b]).wait()
            pltpu.make_async_copy(al_h.at[pl.ds(cbase(i), ch)], abuf[b],
                                  se[b]).wait()

        def compute(b):
            def ce(e, _):
                ex = jnp.exp(abuf[b][e] - cbuf[b][e])
                mbuf[e, pl.ds(QW, 16)] = ex
                for j in range(2):
                    bc = _perm(ex, jnp.full((16,), h2 + j, jnp.int32))
                    for t in range(3):
                        kk = j * 3 + t
                        sl = pl.ds(kk * 16, 16)
                        mbuf[e, sl] = g1[b][e, sl] * bc
                return 0
            lax.fori_loop(0, ch, ce, 0)

        issue_idx(0, 0)
        issue_idx(1, 1)
        wait_idx(0, 0)
        issue_loads(0, 0)

        def step(i, b, ob):
            wait_loads(i, b)

            @pl.when(i + 1 < n)
            def _():
                wait_idx(i + 1, ob)
                issue_loads(i + 1, ob)
            compute(b)
            pltpu.sync_copy(mbuf, acc.at[idx_d[b]], add=True)

            @pl.when(i + 2 < n)
            def _():
                issue_idx(i + 2, b)

        def pair(g, _):
            step(2 * g, 0, 1)
            step(2 * g + 1, 1, 0)
            return 0
        lax.fori_loop(0, n // 2, pair, 0)
        if n % 2:
            step(n - 1, 0, 1)
        _sc_epilogue(acc, out_h, c, s, zbuf, _WD)

    return k(xlq, alpha, c16, src, dst)


def _center_gather_sc(h4, x_pad, center_id):
    ch = C // 32
    mesh = plsc.VectorSubcoreMesh(**_SC_MESH)

    @functools.partial(
        pl.kernel, mesh=mesh,
        compiler_params=pltpu.CompilerParams(use_tc_tiling_on_sc=False),
        interpret=_INTERP,
        out_type=[jax.ShapeDtypeStruct((C, HID), jnp.float32),
                  jax.ShapeDtypeStruct((C, NP), jnp.float32)],
        scratch_types=[
            pltpu.VMEM((ch,), jnp.int32),
            pltpu.VMEM((ch, HID), jnp.float32),
            pltpu.VMEM((ch, NP), jnp.float32),
            pltpu.SemaphoreType.DMA,
        ],
    )
    def k(h4_h, xp_h, cid_h, cv_h, xg_h, idx, buf1, buf2, sem):
        c = lax.axis_index("c")
        s = lax.axis_index("s")
        base = (c * 16 + s) * ch
        pltpu.sync_copy(cid_h.at[pl.ds(base, ch)], idx)
        pltpu.async_copy(h4_h.at[idx], buf1, sem).wait()
        pltpu.sync_copy(buf1, cv_h.at[pl.ds(base, ch)])
        pltpu.async_copy(xp_h.at[idx], buf2, sem).wait()
        pltpu.sync_copy(buf2, xg_h.at[pl.ds(base, ch)])

    return k(h4, x_pad, center_id)


def kernel(x, edge_attr, global_feat, params, edge_index, center_id):
    p = params
    src = edge_index[0]
    dst = edge_index[1]
    x_pad = jnp.pad(x, ((0, 0), (0, NP - NF)))

    fw = _wfuse(p)
    el0, el1q, el2q, eef = _edge_mm(edge_attr, p, fw)

    agg0 = _gine_seg_sum(x_pad, el0[None], src, dst, NP, qcall=None)
    h, h0, h1, h2, h3 = _node_update(0, x_pad, agg0, p)

    for i, elq in ((1, el1q), (2, el2q)):
        hq = jnp.concatenate([h0, h1, h2, h3], axis=0)
        qa = _gine_seg_sum(hq, elq, src, dst, QW, qcall=0)
        qb = _gine_seg_sum(hq, elq, src, dst, QW, qcall=1)
        h, h0, h1, h2, h3 = _node_update(i, h, (qa, qb), p)

    xlf, xrf, x0, x1, x2, x3 = _attn_proj(h, p)
    alpha16, den0 = _gat_alpha_sc(xlf, xrf, eef, src, dst,
                                  p['at_att'].reshape(-1))
    c16 = _cden(den0[0] + den0[1])
    xlq = jnp.concatenate([x0, x1, x2, x3], axis=0)
    gA = _gat_agg_sc(xlq, alpha16, c16, src, dst, qcall=0)
    gB = _gat_agg_sc(xlq, alpha16, c16, src, dst, qcall=1)
    h4 = _gat_finish(h, gA, gB, p)

    cv, xg = _center_gather_sc(h4, x_pad, center_id)
    out, mu, logvar = _head(cv, xg, global_feat, p)
    return ((out[:, 0], out[:, 1]), (mu, logvar))
